# Initial kernel scaffold; baseline (speedup 1.0000x reference)
import jax
import jax.numpy as jnp
from jax.experimental import pallas as pl


def kernel(x, edge_index, W1, b1, W2, b2):
    raise NotImplementedError("write your pallas kernel here")


# speedup vs baseline: 19.1563x; 19.1563x over previous
import functools

import jax
import jax.numpy as jnp
from jax import lax
from jax.experimental import pallas as pl
from jax.experimental.pallas import tpu as pltpu, tpu_sc as plsc

N_NODES = 10000
N_PAD = 10240
N_EDGES = 320000
D = 128
NC, NS = 2, 16
NW = NC * NS
E_PER_W = N_EDGES // NW
BLK = 80
NB = E_PER_W // BLK
ROWS_PER_TILE = N_PAD // NS

_mesh = plsc.VectorSubcoreMesh(
    core_axis_name="c", subcore_axis_name="s", num_cores=NC, num_subcores=NS)


@functools.partial(
    pl.kernel,
    out_type=jax.ShapeDtypeStruct((NC, N_PAD), jnp.float32),
    mesh=_mesh,
    scratch_types=[
        pltpu.VMEM((NB, BLK), jnp.int32),
        pltpu.VMEM((BLK,), jnp.float32),
        pltpu.VMEM((ROWS_PER_TILE,), jnp.float32),
        pltpu.VMEM_SHARED((N_PAD,), jnp.float32),
    ],
)
def _sc_degree(dst_hbm, deg_hbm, idx_v, ones_v, zero_v, deg_sh):
    c = lax.axis_index("c")
    s = lax.axis_index("s")
    wid = s * NC + c

    for j in range(BLK // 16):
        ones_v[pl.ds(j * 16, 16)] = jnp.full((16,), 1.0, jnp.float32)
    for j in range(ROWS_PER_TILE // 16):
        zero_v[pl.ds(j * 16, 16)] = jnp.zeros((16,), jnp.float32)
    pltpu.sync_copy(zero_v, deg_sh.at[pl.ds(s * ROWS_PER_TILE, ROWS_PER_TILE)])
    plsc.subcore_barrier()

    pltpu.sync_copy(dst_hbm.at[wid], idx_v)

    def body(b, _):
        pltpu.sync_copy(ones_v, deg_sh.at[idx_v.at[b]], add=True)
        return ()

    lax.fori_loop(0, NB, body, (), unroll=False)
    plsc.subcore_barrier()
    pltpu.sync_copy(deg_sh.at[pl.ds(s * ROWS_PER_TILE, ROWS_PER_TILE)],
                    deg_hbm.at[c, pl.ds(s * ROWS_PER_TILE, ROWS_PER_TILE)])


@functools.partial(
    pl.kernel,
    out_type=jax.ShapeDtypeStruct((NC, N_PAD, D), jnp.float32),
    mesh=_mesh,
    scratch_types=[
        pltpu.VMEM((NB, BLK), jnp.int32),
        pltpu.VMEM((NB, BLK), jnp.int32),
        pltpu.VMEM((BLK, D), jnp.float32),
        pltpu.VMEM_SHARED((N_PAD, D), jnp.float32),
        pltpu.SemaphoreType.DMA,
    ],
)
def _sc_aggregate(src_hbm, dst_hbm, hs_hbm, out_hbm,
                  isrc_v, idst_v, rows_v, agg_sh, sem):
    c = lax.axis_index("c")
    s = lax.axis_index("s")
    wid = s * NC + c
    row0 = s * ROWS_PER_TILE

    def zero_body(r, _):
        for l in range(D // 16):
            rows_v[r, pl.ds(l * 16, 16)] = jnp.zeros((16,), jnp.float32)
        return ()

    lax.fori_loop(0, BLK, zero_body, (), unroll=False)
    for j in range(ROWS_PER_TILE // BLK):
        pltpu.sync_copy(rows_v, agg_sh.at[pl.ds(row0 + j * BLK, BLK)])
    plsc.subcore_barrier()

    pltpu.sync_copy(src_hbm.at[wid], isrc_v)
    pltpu.sync_copy(dst_hbm.at[wid], idst_v)

    def body(b, _):
        pltpu.async_copy(hs_hbm.at[isrc_v.at[b]], rows_v, sem).wait()
        pltpu.sync_copy(rows_v, agg_sh.at[idst_v.at[b]], add=True)
        return ()

    lax.fori_loop(0, NB, body, (), unroll=False)
    plsc.subcore_barrier()
    pltpu.sync_copy(agg_sh.at[pl.ds(row0, ROWS_PER_TILE)],
                    out_hbm.at[c, pl.ds(row0, ROWS_PER_TILE)])


_TC_ROWS = 1024
_TC_GRID = N_PAD // _TC_ROWS

_row_spec = pl.BlockSpec((_TC_ROWS, D), lambda i: (i, 0))
_w_spec = pl.BlockSpec((D, D), lambda i: (0, 0))
_b_spec = pl.BlockSpec((1, D), lambda i: (0, 0))


def _tc1_body(x_ref, w_ref, d0_ref, d1_ref, hs_ref, dinv_ref):
    deg = d0_ref[...] + d1_ref[...] + 1.0
    dinv = lax.rsqrt(deg)
    h = jnp.dot(x_ref[...], w_ref[...], preferred_element_type=jnp.float32)
    hs_ref[...] = h * dinv
    dinv_ref[...] = dinv


def _tc1(x, w1, d0b, d1b):
    return pl.pallas_call(
        _tc1_body,
        grid=(_TC_GRID,),
        in_specs=[_row_spec, _w_spec, _row_spec, _row_spec],
        out_specs=[_row_spec, _row_spec],
        out_shape=[jax.ShapeDtypeStruct((N_PAD, D), jnp.float32),
                   jax.ShapeDtypeStruct((N_PAD, D), jnp.float32)],
    )(x, w1, d0b, d1b)


def _tc2_body(p0_ref, p1_ref, hs1_ref, dinv_ref, b1_ref, w_ref, hs2_ref):
    dinv = dinv_ref[...]
    z = (p0_ref[...] + p1_ref[...] + hs1_ref[...]) * dinv + b1_ref[...]
    z = jnp.maximum(z, 0.0)
    h2 = jnp.dot(z, w_ref[...], preferred_element_type=jnp.float32)
    hs2_ref[...] = h2 * dinv


def _tc2(p0, p1, hs1, dinvb, b1, w2):
    return pl.pallas_call(
        _tc2_body,
        grid=(_TC_GRID,),
        in_specs=[_row_spec, _row_spec, _row_spec, _row_spec, _b_spec, _w_spec],
        out_specs=_row_spec,
        out_shape=jax.ShapeDtypeStruct((N_PAD, D), jnp.float32),
    )(p0, p1, hs1, dinvb, b1, w2)


def _tc3_body(q0_ref, q1_ref, hs2_ref, dinv_ref, b2_ref, out_ref):
    out_ref[...] = ((q0_ref[...] + q1_ref[...] + hs2_ref[...]) * dinv_ref[...]
                    + b2_ref[...])


def _tc3(q0, q1, hs2, dinvb, b2):
    return pl.pallas_call(
        _tc3_body,
        grid=(_TC_GRID,),
        in_specs=[_row_spec, _row_spec, _row_spec, _row_spec, _b_spec],
        out_specs=_row_spec,
        out_shape=jax.ShapeDtypeStruct((N_PAD, D), jnp.float32),
    )(q0, q1, hs2, dinvb, b2)


def kernel(x, edge_index, W1, b1, W2, b2):
    e = edge_index.astype(jnp.int32)
    src3 = e[0].reshape(NW, NB, BLK)
    dst3 = e[1].reshape(NW, NB, BLK)
    x_pad = jnp.pad(x, ((0, N_PAD - N_NODES), (0, 0)))
    b1r = b1.reshape(1, D)
    b2r = b2.reshape(1, D)

    degp = _sc_degree(dst3)
    d0b = jnp.broadcast_to(degp[0][:, None], (N_PAD, D))
    d1b = jnp.broadcast_to(degp[1][:, None], (N_PAD, D))

    hs1, dinvb = _tc1(x_pad, W1, d0b, d1b)
    p = _sc_aggregate(src3, dst3, hs1)
    hs2 = _tc2(p[0], p[1], hs1, dinvb, b1r, W2)
    q = _sc_aggregate(src3, dst3, hs2)
    out = _tc3(q[0], q[1], hs2, dinvb, b2r)
    return out[:N_NODES]


# speedup vs baseline: 25.1994x; 1.3155x over previous
import functools

import jax
import jax.numpy as jnp
from jax import lax
from jax.experimental import pallas as pl
from jax.experimental.pallas import tpu as pltpu, tpu_sc as plsc

N_NODES = 10000
DEG_PAD = 10240
N_EDGES = 320000
D = 128
NC, NS = 2, 16
NW = NC * NS
E_PER_W = N_EDGES // NW

_mesh = plsc.VectorSubcoreMesh(
    core_axis_name="c", subcore_axis_name="s", num_cores=NC, num_subcores=NS)


DBLK = 80
DNB = E_PER_W // DBLK
DEG_ROWS_PER_TILE = DEG_PAD // NS


@functools.partial(
    pl.kernel,
    out_type=jax.ShapeDtypeStruct((NC, DEG_PAD), jnp.float32),
    mesh=_mesh,
    scratch_types=[
        pltpu.VMEM((DNB, DBLK), jnp.int32),
        pltpu.VMEM((DBLK,), jnp.float32),
        pltpu.VMEM((DEG_ROWS_PER_TILE,), jnp.float32),
        pltpu.VMEM_SHARED((DEG_PAD,), jnp.float32),
    ],
)
def _sc_degree(dst_hbm, deg_hbm, idx_v, ones_v, zero_v, deg_sh):
    c = lax.axis_index("c")
    s = lax.axis_index("s")
    wid = s * NC + c

    for j in range(DBLK // 16):
        ones_v[pl.ds(j * 16, 16)] = jnp.full((16,), 1.0, jnp.float32)
    for j in range(DEG_ROWS_PER_TILE // 16):
        zero_v[pl.ds(j * 16, 16)] = jnp.zeros((16,), jnp.float32)
    pltpu.sync_copy(zero_v,
                    deg_sh.at[pl.ds(s * DEG_ROWS_PER_TILE, DEG_ROWS_PER_TILE)])
    plsc.subcore_barrier()

    pltpu.sync_copy(dst_hbm.at[wid], idx_v)

    def body(b, _):
        pltpu.sync_copy(ones_v, deg_sh.at[idx_v.at[b]], add=True)
        return ()

    lax.fori_loop(0, DNB, body, (), unroll=False)
    plsc.subcore_barrier()
    pltpu.sync_copy(deg_sh.at[pl.ds(s * DEG_ROWS_PER_TILE, DEG_ROWS_PER_TILE)],
                    deg_hbm.at[c, pl.ds(s * DEG_ROWS_PER_TILE,
                                        DEG_ROWS_PER_TILE)])


BLK = 125
NB = E_PER_W // BLK
NBUF = 2
WAVES = NB // NBUF
WPAIR = WAVES // 2
ROWS_PER_TILE = DEG_PAD // NS


@functools.partial(
    pl.kernel,
    out_type=jax.ShapeDtypeStruct((NC, DEG_PAD, D), jnp.float32),
    mesh=_mesh,
    scratch_types=[
        pltpu.VMEM((NB, BLK), jnp.int32),
        pltpu.VMEM((NBUF, BLK), jnp.int32),
        pltpu.VMEM((NBUF, BLK), jnp.int32),
        pltpu.VMEM((NBUF, BLK, D), jnp.float32),
        pltpu.VMEM_SHARED((DEG_PAD, D), jnp.float32),
        [pltpu.SemaphoreType.DMA] * NBUF,
        [pltpu.SemaphoreType.DMA] * NBUF,
        pltpu.SemaphoreType.DMA,
        pltpu.SemaphoreType.DMA,
    ],
)
def _sc_aggregate(src_hbm, dst_hbm, hs_hbm, out_hbm,
                  idst_v, ixa_v, ixb_v, rows_v, agg_sh,
                  gsem, ssem, isema, isemb):
    c = lax.axis_index("c")
    s = lax.axis_index("s")
    wid = s * NC + c
    row0 = s * ROWS_PER_TILE

    for r in range(8):
        for l in range(D // 16):
            rows_v[0, r, pl.ds(l * 16, 16)] = jnp.zeros((16,), jnp.float32)

    def zero_body(t, _):
        pltpu.sync_copy(rows_v.at[0, pl.ds(0, 8)],
                        agg_sh.at[pl.ds(row0 + t * 8, 8)])
        return ()

    lax.fori_loop(0, ROWS_PER_TILE // 8, zero_body, (), unroll=False)
    plsc.subcore_barrier()

    pltpu.sync_copy(dst_hbm.at[wid], idst_v)
    pltpu.async_copy(src_hbm.at[wid, 0], ixa_v, isema)
    pltpu.async_copy(src_hbm.at[wid, 1], ixb_v, isemb)
    pltpu.make_async_copy(src_hbm.at[wid, 0], ixa_v, isema).wait()
    for j in range(NBUF):
        pltpu.async_copy(hs_hbm.at[ixa_v.at[j]], rows_v.at[j], gsem[j])

    def body(k, _):
        for j in range(NBUF):
            b = 2 * k * NBUF + j
            pltpu.make_async_copy(hs_hbm.at[ixa_v.at[j]], rows_v.at[j],
                                  gsem[j]).wait()
            pltpu.async_copy(rows_v.at[j], agg_sh.at[idst_v.at[b]], ssem[j],
                             add=True)

        @pl.when(k < WPAIR - 1)
        def _():
            pltpu.async_copy(src_hbm.at[wid, 2 * k + 2], ixa_v, isema)

        pltpu.make_async_copy(src_hbm.at[wid, 0], ixb_v, isemb).wait()
        for j in range(NBUF):
            pltpu.make_async_copy(rows_v.at[j], agg_sh.at[idst_v.at[0]],
                                  ssem[j]).wait()
            pltpu.async_copy(hs_hbm.at[ixb_v.at[j]], rows_v.at[j], gsem[j])

        for j in range(NBUF):
            b = (2 * k + 1) * NBUF + j
            pltpu.make_async_copy(hs_hbm.at[ixb_v.at[j]], rows_v.at[j],
                                  gsem[j]).wait()
            pltpu.async_copy(rows_v.at[j], agg_sh.at[idst_v.at[b]], ssem[j],
                             add=True)

        @pl.when(k < WPAIR - 1)
        def _():
            pltpu.async_copy(src_hbm.at[wid, 2 * k + 3], ixb_v, isemb)

        @pl.when(k < WPAIR - 1)
        def _():
            pltpu.make_async_copy(src_hbm.at[wid, 0], ixa_v, isema).wait()
        for j in range(NBUF):
            pltpu.make_async_copy(rows_v.at[j], agg_sh.at[idst_v.at[0]],
                                  ssem[j]).wait()

            @pl.when(k < WPAIR - 1)
            def _():
                pltpu.async_copy(hs_hbm.at[ixa_v.at[j]], rows_v.at[j], gsem[j])

        return ()

    lax.fori_loop(0, WPAIR, body, (), unroll=False)
    plsc.subcore_barrier()
    pltpu.sync_copy(agg_sh.at[pl.ds(row0, ROWS_PER_TILE)],
                    out_hbm.at[c, pl.ds(row0, ROWS_PER_TILE)])


_TC_ROWS = 1000
_TC_GRID = N_NODES // _TC_ROWS

_row_spec = pl.BlockSpec((_TC_ROWS, D), lambda i: (i, 0))
_w_spec = pl.BlockSpec((D, D), lambda i: (0, 0))
_b_spec = pl.BlockSpec((1, D), lambda i: (0, 0))


def _tc1_body(x_ref, w_ref, d0_ref, d1_ref, hs_ref, dinv_ref):
    deg = d0_ref[...] + d1_ref[...] + 1.0
    dinv = lax.rsqrt(deg)
    h = jnp.dot(x_ref[...], w_ref[...], preferred_element_type=jnp.float32)
    hs_ref[...] = h * dinv
    dinv_ref[...] = dinv


def _tc1(x, w1, d0b, d1b):
    return pl.pallas_call(
        _tc1_body,
        grid=(_TC_GRID,),
        in_specs=[_row_spec, _w_spec, _row_spec, _row_spec],
        out_specs=[_row_spec, _row_spec],
        out_shape=[jax.ShapeDtypeStruct((N_NODES, D), jnp.float32),
                   jax.ShapeDtypeStruct((N_NODES, D), jnp.float32)],
    )(x, w1, d0b, d1b)


def _tc2_body(p0_ref, p1_ref, hs1_ref, dinv_ref, b1_ref, w_ref, hs2_ref):
    dinv = dinv_ref[...]
    z = (p0_ref[...] + p1_ref[...] + hs1_ref[...]) * dinv + b1_ref[...]
    z = jnp.maximum(z, 0.0)
    h2 = jnp.dot(z, w_ref[...], preferred_element_type=jnp.float32)
    hs2_ref[...] = h2 * dinv


def _tc2(p0, p1, hs1, dinvb, b1, w2):
    return pl.pallas_call(
        _tc2_body,
        grid=(_TC_GRID,),
        in_specs=[_row_spec, _row_spec, _row_spec, _row_spec, _b_spec, _w_spec],
        out_specs=_row_spec,
        out_shape=jax.ShapeDtypeStruct((N_NODES, D), jnp.float32),
    )(p0, p1, hs1, dinvb, b1, w2)


def _tc3_body(q0_ref, q1_ref, hs2_ref, dinv_ref, b2_ref, out_ref):
    out_ref[...] = ((q0_ref[...] + q1_ref[...] + hs2_ref[...]) * dinv_ref[...]
                    + b2_ref[...])


def _tc3(q0, q1, hs2, dinvb, b2):
    return pl.pallas_call(
        _tc3_body,
        grid=(_TC_GRID,),
        in_specs=[_row_spec, _row_spec, _row_spec, _row_spec, _b_spec],
        out_specs=_row_spec,
        out_shape=jax.ShapeDtypeStruct((N_NODES, D), jnp.float32),
    )(q0, q1, hs2, dinvb, b2)


def kernel(x, edge_index, W1, b1, W2, b2):
    e = edge_index.astype(jnp.int32)
    src4 = e[0].reshape(NW, WAVES, NBUF, BLK)
    dst3 = e[1].reshape(NW, NB, BLK)
    dstd = e[1].reshape(NW, DNB, DBLK)
    b1r = b1.reshape(1, D)
    b2r = b2.reshape(1, D)

    degp = _sc_degree(dstd)
    d0b = jnp.broadcast_to(degp[0, :N_NODES][:, None], (N_NODES, D))
    d1b = jnp.broadcast_to(degp[1, :N_NODES][:, None], (N_NODES, D))

    hs1, dinvb = _tc1(x, W1, d0b, d1b)
    p = _sc_aggregate(src4, dst3, hs1)
    hs2 = _tc2(p[0, :N_NODES], p[1, :N_NODES], hs1, dinvb, b1r, W2)
    q = _sc_aggregate(src4, dst3, hs2)
    return _tc3(q[0, :N_NODES], q[1, :N_NODES], hs2, dinvb, b2r)


# speedup vs baseline: 28.5596x; 1.1333x over previous
import functools

import jax
import jax.numpy as jnp
from jax import lax
from jax.experimental import pallas as pl
from jax.experimental.pallas import tpu as pltpu, tpu_sc as plsc

N_NODES = 10000
DEG_PAD = 10240
N_EDGES = 320000
D = 128
NC, NS = 2, 16
NW = NC * NS
E_PER_W = N_EDGES // NW

_mesh = plsc.VectorSubcoreMesh(
    core_axis_name="c", subcore_axis_name="s", num_cores=NC, num_subcores=NS)


DBLK = 80
DNB = E_PER_W // DBLK
DEG_ROWS_PER_TILE = DEG_PAD // NS


@functools.partial(
    pl.kernel,
    out_type=jax.ShapeDtypeStruct((NC, DEG_PAD), jnp.float32),
    mesh=_mesh,
    scratch_types=[
        pltpu.VMEM((DNB, DBLK), jnp.int32),
        pltpu.VMEM((DBLK,), jnp.float32),
        pltpu.VMEM((DEG_ROWS_PER_TILE,), jnp.float32),
        pltpu.VMEM_SHARED((DEG_PAD,), jnp.float32),
    ],
)
def _sc_degree(dst_hbm, deg_hbm, idx_v, ones_v, zero_v, deg_sh):
    c = lax.axis_index("c")
    s = lax.axis_index("s")
    wid = s * NC + c

    for j in range(DBLK // 16):
        ones_v[pl.ds(j * 16, 16)] = jnp.full((16,), 1.0, jnp.float32)
    for j in range(DEG_ROWS_PER_TILE // 16):
        zero_v[pl.ds(j * 16, 16)] = jnp.zeros((16,), jnp.float32)
    pltpu.sync_copy(zero_v,
                    deg_sh.at[pl.ds(s * DEG_ROWS_PER_TILE, DEG_ROWS_PER_TILE)])
    plsc.subcore_barrier()

    pltpu.sync_copy(dst_hbm.at[wid], idx_v)

    def body(b, _):
        pltpu.sync_copy(ones_v, deg_sh.at[idx_v.at[b]], add=True)
        return ()

    lax.fori_loop(0, DNB, body, (), unroll=False)
    plsc.subcore_barrier()
    pltpu.sync_copy(deg_sh.at[pl.ds(s * DEG_ROWS_PER_TILE, DEG_ROWS_PER_TILE)],
                    deg_hbm.at[c, pl.ds(s * DEG_ROWS_PER_TILE,
                                        DEG_ROWS_PER_TILE)])


BLK = 40
NB = E_PER_W // BLK
NBUF = 5
WAVES = NB // NBUF
WPAIR = WAVES // 2
ROWS_PER_TILE = DEG_PAD // NS


@functools.partial(
    pl.kernel,
    out_type=jax.ShapeDtypeStruct((NC, DEG_PAD, D), jnp.float32),
    mesh=_mesh,
    scratch_types=[
        pltpu.VMEM((2, NBUF, BLK), jnp.int32),
        pltpu.VMEM((2, NBUF, BLK), jnp.int32),
        pltpu.VMEM((NBUF, BLK, D), jnp.float32),
        pltpu.VMEM_SHARED((DEG_PAD, D), jnp.float32),
        [pltpu.SemaphoreType.DMA] * NBUF,
        [pltpu.SemaphoreType.DMA] * NBUF,
        pltpu.SemaphoreType.DMA,
        pltpu.SemaphoreType.DMA,
    ],
)
def _sc_aggregate(idx_hbm, hs_hbm, out_hbm,
                  ixa_v, ixb_v, rows_v, agg_sh,
                  gsem, ssem, isema, isemb):
    c = lax.axis_index("c")
    s = lax.axis_index("s")
    wid = s * NC + c
    row0 = s * ROWS_PER_TILE

    for r in range(8):
        for l in range(D // 16):
            rows_v[0, r, pl.ds(l * 16, 16)] = jnp.zeros((16,), jnp.float32)

    def zero_body(t, _):
        pltpu.sync_copy(rows_v.at[0, pl.ds(0, 8)],
                        agg_sh.at[pl.ds(row0 + t * 8, 8)])
        return ()

    lax.fori_loop(0, ROWS_PER_TILE // 8, zero_body, (), unroll=False)
    plsc.subcore_barrier()

    pltpu.async_copy(idx_hbm.at[wid, 0], ixa_v, isema)
    pltpu.async_copy(idx_hbm.at[wid, 1], ixb_v, isemb)
    pltpu.make_async_copy(idx_hbm.at[wid, 0], ixa_v, isema).wait()
    for j in range(NBUF):
        pltpu.async_copy(hs_hbm.at[ixa_v.at[0, j]], rows_v.at[j], gsem[j])

    def body(k, _):
        for j in range(NBUF):
            pltpu.make_async_copy(hs_hbm.at[ixa_v.at[0, j]], rows_v.at[j],
                                  gsem[j]).wait()
            pltpu.async_copy(rows_v.at[j], agg_sh.at[ixa_v.at[1, j]], ssem[j],
                             add=True)

        pltpu.make_async_copy(idx_hbm.at[wid, 0], ixb_v, isemb).wait()
        for j in range(NBUF):
            pltpu.make_async_copy(rows_v.at[j], agg_sh.at[ixa_v.at[1, j]],
                                  ssem[j]).wait()
            pltpu.async_copy(hs_hbm.at[ixb_v.at[0, j]], rows_v.at[j], gsem[j])

        @pl.when(k < WPAIR - 1)
        def _():
            pltpu.async_copy(idx_hbm.at[wid, 2 * k + 2], ixa_v, isema)

        for j in range(NBUF):
            pltpu.make_async_copy(hs_hbm.at[ixb_v.at[0, j]], rows_v.at[j],
                                  gsem[j]).wait()
            pltpu.async_copy(rows_v.at[j], agg_sh.at[ixb_v.at[1, j]], ssem[j],
                             add=True)

        @pl.when(k < WPAIR - 1)
        def _():
            pltpu.make_async_copy(idx_hbm.at[wid, 0], ixa_v, isema).wait()
        for j in range(NBUF):
            pltpu.make_async_copy(rows_v.at[j], agg_sh.at[ixb_v.at[1, j]],
                                  ssem[j]).wait()

            @pl.when(k < WPAIR - 1)
            def _():
                pltpu.async_copy(hs_hbm.at[ixa_v.at[0, j]], rows_v.at[j],
                                 gsem[j])

        @pl.when(k < WPAIR - 1)
        def _():
            pltpu.async_copy(idx_hbm.at[wid, 2 * k + 3], ixb_v, isemb)

        return ()

    lax.fori_loop(0, WPAIR, body, (), unroll=False)
    plsc.subcore_barrier()
    pltpu.sync_copy(agg_sh.at[pl.ds(row0, ROWS_PER_TILE)],
                    out_hbm.at[c, pl.ds(row0, ROWS_PER_TILE)])


_TC_ROWS = 1000
_TC_GRID = N_NODES // _TC_ROWS

_row_spec = pl.BlockSpec((_TC_ROWS, D), lambda i: (i, 0))
_w_spec = pl.BlockSpec((D, D), lambda i: (0, 0))
_b_spec = pl.BlockSpec((1, D), lambda i: (0, 0))


def _tc1_body(x_ref, w_ref, d0_ref, d1_ref, hs_ref, dinv_ref):
    deg = d0_ref[...] + d1_ref[...] + 1.0
    dinv = lax.rsqrt(deg)
    h = jnp.dot(x_ref[...], w_ref[...], preferred_element_type=jnp.float32)
    hs_ref[...] = h * dinv
    dinv_ref[...] = dinv


def _tc1(x, w1, d0b, d1b):
    return pl.pallas_call(
        _tc1_body,
        grid=(_TC_GRID,),
        in_specs=[_row_spec, _w_spec, _row_spec, _row_spec],
        out_specs=[_row_spec, _row_spec],
        out_shape=[jax.ShapeDtypeStruct((N_NODES, D), jnp.float32),
                   jax.ShapeDtypeStruct((N_NODES, D), jnp.float32)],
    )(x, w1, d0b, d1b)


def _tc2_body(p0_ref, p1_ref, hs1_ref, dinv_ref, b1_ref, w_ref, hs2_ref):
    dinv = dinv_ref[...]
    z = (p0_ref[...] + p1_ref[...] + hs1_ref[...]) * dinv + b1_ref[...]
    z = jnp.maximum(z, 0.0)
    h2 = jnp.dot(z, w_ref[...], preferred_element_type=jnp.float32)
    hs2_ref[...] = h2 * dinv


def _tc2(p0, p1, hs1, dinvb, b1, w2):
    return pl.pallas_call(
        _tc2_body,
        grid=(_TC_GRID,),
        in_specs=[_row_spec, _row_spec, _row_spec, _row_spec, _b_spec, _w_spec],
        out_specs=_row_spec,
        out_shape=jax.ShapeDtypeStruct((N_NODES, D), jnp.float32),
    )(p0, p1, hs1, dinvb, b1, w2)


def _tc3_body(q0_ref, q1_ref, hs2_ref, dinv_ref, b2_ref, out_ref):
    out_ref[...] = ((q0_ref[...] + q1_ref[...] + hs2_ref[...]) * dinv_ref[...]
                    + b2_ref[...])


def _tc3(q0, q1, hs2, dinvb, b2):
    return pl.pallas_call(
        _tc3_body,
        grid=(_TC_GRID,),
        in_specs=[_row_spec, _row_spec, _row_spec, _row_spec, _b_spec],
        out_specs=_row_spec,
        out_shape=jax.ShapeDtypeStruct((N_NODES, D), jnp.float32),
    )(q0, q1, hs2, dinvb, b2)


def kernel(x, edge_index, W1, b1, W2, b2):
    e = edge_index.astype(jnp.int32)
    srcr = e[0].reshape(NW, WAVES, 1, NBUF, BLK)
    dstr = e[1].reshape(NW, WAVES, 1, NBUF, BLK)
    ei = jnp.concatenate([srcr, dstr], axis=2)
    dstd = e[1].reshape(NW, DNB, DBLK)
    b1r = b1.reshape(1, D)
    b2r = b2.reshape(1, D)

    degp = _sc_degree(dstd)
    d0b = jnp.broadcast_to(degp[0, :N_NODES][:, None], (N_NODES, D))
    d1b = jnp.broadcast_to(degp[1, :N_NODES][:, None], (N_NODES, D))

    hs1, dinvb = _tc1(x, W1, d0b, d1b)
    p = _sc_aggregate(ei, hs1)
    hs2 = _tc2(p[0, :N_NODES], p[1, :N_NODES], hs1, dinvb, b1r, W2)
    q = _sc_aggregate(ei, hs2)
    return _tc3(q[0, :N_NODES], q[1, :N_NODES], hs2, dinvb, b2r)


# speedup vs baseline: 28.9864x; 1.0149x over previous
import functools

import jax
import jax.numpy as jnp
from jax import lax
from jax.experimental import pallas as pl
from jax.experimental.pallas import tpu as pltpu, tpu_sc as plsc

N_NODES = 10000
DEG_PAD = 10240
N_EDGES = 320000
D = 128
NC, NS = 2, 16
NW = NC * NS
E_PER_W = N_EDGES // NW

_mesh = plsc.VectorSubcoreMesh(
    core_axis_name="c", subcore_axis_name="s", num_cores=NC, num_subcores=NS)


DBLK = 80
DNB = E_PER_W // DBLK
DEG_ROWS_PER_TILE = DEG_PAD // NS


@functools.partial(
    pl.kernel,
    out_type=jax.ShapeDtypeStruct((NC, DEG_PAD), jnp.float32),
    mesh=_mesh,
    scratch_types=[
        pltpu.VMEM((DNB, DBLK), jnp.int32),
        pltpu.VMEM((DBLK,), jnp.float32),
        pltpu.VMEM((DEG_ROWS_PER_TILE,), jnp.float32),
        pltpu.VMEM_SHARED((DEG_PAD,), jnp.float32),
    ],
)
def _sc_degree(dst_hbm, deg_hbm, idx_v, ones_v, zero_v, deg_sh):
    c = lax.axis_index("c")
    s = lax.axis_index("s")
    wid = s * NC + c

    for j in range(DBLK // 16):
        ones_v[pl.ds(j * 16, 16)] = jnp.full((16,), 1.0, jnp.float32)
    for j in range(DEG_ROWS_PER_TILE // 16):
        zero_v[pl.ds(j * 16, 16)] = jnp.zeros((16,), jnp.float32)
    pltpu.sync_copy(zero_v,
                    deg_sh.at[pl.ds(s * DEG_ROWS_PER_TILE, DEG_ROWS_PER_TILE)])
    plsc.subcore_barrier()

    pltpu.sync_copy(dst_hbm.at[wid], idx_v)

    def body(b, _):
        pltpu.sync_copy(ones_v, deg_sh.at[idx_v.at[b]], add=True)
        return ()

    lax.fori_loop(0, DNB, body, (), unroll=False)
    plsc.subcore_barrier()
    pltpu.sync_copy(deg_sh.at[pl.ds(s * DEG_ROWS_PER_TILE, DEG_ROWS_PER_TILE)],
                    deg_hbm.at[c, pl.ds(s * DEG_ROWS_PER_TILE,
                                        DEG_ROWS_PER_TILE)])


BLK = 40
NB = E_PER_W // BLK
NBUF = 5
WAVES = NB // NBUF
WPAIR = WAVES // 2
ROWS_PER_TILE = DEG_PAD // NS


@functools.partial(
    pl.kernel,
    out_type=jax.ShapeDtypeStruct((NC, DEG_PAD, D), jnp.float32),
    mesh=_mesh,
    scratch_types=[
        pltpu.VMEM((2, NBUF, BLK), jnp.int32),
        pltpu.VMEM((2, NBUF, BLK), jnp.int32),
        pltpu.VMEM((NBUF, BLK, D), jnp.float32),
        pltpu.VMEM_SHARED((DEG_PAD, D), jnp.float32),
        [pltpu.SemaphoreType.DMA] * NBUF,
        [pltpu.SemaphoreType.DMA] * NBUF,
        pltpu.SemaphoreType.DMA,
        pltpu.SemaphoreType.DMA,
    ],
)
def _sc_aggregate(idx_hbm, hs_hbm, out_hbm,
                  ixa_v, ixb_v, rows_v, agg_sh,
                  gsem, ssem, isema, isemb):
    c = lax.axis_index("c")
    s = lax.axis_index("s")
    wid = s * NC + c
    row0 = s * ROWS_PER_TILE

    for r in range(8):
        for l in range(D // 16):
            rows_v[0, r, pl.ds(l * 16, 16)] = jnp.zeros((16,), jnp.float32)

    def zero_body(t, _):
        pltpu.sync_copy(rows_v.at[0, pl.ds(0, 8)],
                        agg_sh.at[pl.ds(row0 + t * 8, 8)])
        return ()

    lax.fori_loop(0, ROWS_PER_TILE // 8, zero_body, (), unroll=False)
    plsc.subcore_barrier()

    pltpu.async_copy(idx_hbm.at[wid, 0], ixa_v, isema)
    pltpu.async_copy(idx_hbm.at[wid, 1], ixb_v, isemb)
    pltpu.make_async_copy(idx_hbm.at[wid, 0], ixa_v, isema).wait()
    for j in range(NBUF):
        pltpu.async_copy(hs_hbm.at[ixa_v.at[0, j]], rows_v.at[j], gsem[j])

    def body(k, _):
        for j in range(NBUF):
            pltpu.make_async_copy(hs_hbm.at[ixa_v.at[0, j]], rows_v.at[j],
                                  gsem[j]).wait()
            pltpu.async_copy(rows_v.at[j], agg_sh.at[ixa_v.at[1, j]], ssem[j],
                             add=True)

        pltpu.make_async_copy(idx_hbm.at[wid, 0], ixb_v, isemb).wait()
        for j in range(NBUF):
            pltpu.make_async_copy(rows_v.at[j], agg_sh.at[ixa_v.at[1, j]],
                                  ssem[j]).wait()
            pltpu.async_copy(hs_hbm.at[ixb_v.at[0, j]], rows_v.at[j], gsem[j])

        @pl.when(k < WPAIR - 1)
        def _():
            pltpu.async_copy(idx_hbm.at[wid, 2 * k + 2], ixa_v, isema)

        for j in range(NBUF):
            pltpu.make_async_copy(hs_hbm.at[ixb_v.at[0, j]], rows_v.at[j],
                                  gsem[j]).wait()
            pltpu.async_copy(rows_v.at[j], agg_sh.at[ixb_v.at[1, j]], ssem[j],
                             add=True)

        @pl.when(k < WPAIR - 1)
        def _():
            pltpu.make_async_copy(idx_hbm.at[wid, 0], ixa_v, isema).wait()
        for j in range(NBUF):
            pltpu.make_async_copy(rows_v.at[j], agg_sh.at[ixb_v.at[1, j]],
                                  ssem[j]).wait()

            @pl.when(k < WPAIR - 1)
            def _():
                pltpu.async_copy(hs_hbm.at[ixa_v.at[0, j]], rows_v.at[j],
                                 gsem[j])

        @pl.when(k < WPAIR - 1)
        def _():
            pltpu.async_copy(idx_hbm.at[wid, 2 * k + 3], ixb_v, isemb)

        return ()

    lax.fori_loop(0, WPAIR, body, (), unroll=False)
    plsc.subcore_barrier()
    pltpu.sync_copy(agg_sh.at[pl.ds(row0, ROWS_PER_TILE)],
                    out_hbm.at[c, pl.ds(row0, ROWS_PER_TILE)])


_TC_ROWS = 1000
_TC_GRID = N_NODES // _TC_ROWS

_row_spec = pl.BlockSpec((_TC_ROWS, D), lambda i: (i, 0))
_w_spec = pl.BlockSpec((D, D), lambda i: (0, 0))
_b_spec = pl.BlockSpec((1, D), lambda i: (0, 0))
_p0_spec = pl.BlockSpec((1, _TC_ROWS, D), lambda i: (0, i, 0))
_p1_spec = pl.BlockSpec((1, _TC_ROWS, D), lambda i: (1, i, 0))


def _tc1a_body(x_ref, w_ref, h_ref):
    h_ref[...] = jnp.dot(x_ref[...], w_ref[...],
                         preferred_element_type=jnp.float32)


def _tc1a(x, w1):
    return pl.pallas_call(
        _tc1a_body,
        grid=(_TC_GRID,),
        in_specs=[_row_spec, _w_spec],
        out_specs=_row_spec,
        out_shape=jax.ShapeDtypeStruct((N_NODES, D), jnp.float32),
    )(x, w1)


def _tc1b_body(h_ref, d0_ref, d1_ref, hs_ref, dinv_ref):
    deg = d0_ref[...] + d1_ref[...] + 1.0
    dinv = lax.rsqrt(deg)
    hs_ref[...] = h_ref[...] * dinv
    dinv_ref[...] = dinv


def _tc1b(h1, d0b, d1b):
    return pl.pallas_call(
        _tc1b_body,
        grid=(_TC_GRID,),
        in_specs=[_row_spec, _row_spec, _row_spec],
        out_specs=[_row_spec, _row_spec],
        out_shape=[jax.ShapeDtypeStruct((N_NODES, D), jnp.float32),
                   jax.ShapeDtypeStruct((N_NODES, D), jnp.float32)],
    )(h1, d0b, d1b)


def _tc2_body(p0_ref, p1_ref, hs1_ref, dinv_ref, b1_ref, w_ref, hs2_ref):
    dinv = dinv_ref[...]
    z = ((p0_ref[0] + p1_ref[0] + hs1_ref[...]) * dinv + b1_ref[...])
    z = jnp.maximum(z, 0.0)
    h2 = jnp.dot(z, w_ref[...], preferred_element_type=jnp.float32)
    hs2_ref[...] = h2 * dinv


def _tc2(p, hs1, dinvb, b1, w2):
    return pl.pallas_call(
        _tc2_body,
        grid=(_TC_GRID,),
        in_specs=[_p0_spec, _p1_spec, _row_spec, _row_spec, _b_spec, _w_spec],
        out_specs=_row_spec,
        out_shape=jax.ShapeDtypeStruct((N_NODES, D), jnp.float32),
    )(p, p, hs1, dinvb, b1, w2)


def _tc3_body(q0_ref, q1_ref, hs2_ref, dinv_ref, b2_ref, out_ref):
    out_ref[...] = ((q0_ref[0] + q1_ref[0] + hs2_ref[...]) * dinv_ref[...]
                    + b2_ref[...])


def _tc3(q, hs2, dinvb, b2):
    return pl.pallas_call(
        _tc3_body,
        grid=(_TC_GRID,),
        in_specs=[_p0_spec, _p1_spec, _row_spec, _row_spec, _b_spec],
        out_specs=_row_spec,
        out_shape=jax.ShapeDtypeStruct((N_NODES, D), jnp.float32),
    )(q, q, hs2, dinvb, b2)


def kernel(x, edge_index, W1, b1, W2, b2):
    e = edge_index.astype(jnp.int32)
    srcr = e[0].reshape(NW, WAVES, 1, NBUF, BLK)
    dstr = e[1].reshape(NW, WAVES, 1, NBUF, BLK)
    ei = jnp.concatenate([srcr, dstr], axis=2)
    dstd = e[1].reshape(NW, DNB, DBLK)
    b1r = b1.reshape(1, D)
    b2r = b2.reshape(1, D)

    h1 = _tc1a(x, W1)
    degp = _sc_degree(dstd)
    d0b = jnp.broadcast_to(degp[0, :N_NODES][:, None], (N_NODES, D))
    d1b = jnp.broadcast_to(degp[1, :N_NODES][:, None], (N_NODES, D))

    hs1, dinvb = _tc1b(h1, d0b, d1b)
    p = _sc_aggregate(ei, hs1)
    hs2 = _tc2(p, hs1, dinvb, b1r, W2)
    q = _sc_aggregate(ei, hs2)
    return _tc3(q, hs2, dinvb, b2r)


# speedup vs baseline: 30.9185x; 1.0667x over previous
import functools

import jax
import jax.numpy as jnp
from jax import lax
from jax.experimental import pallas as pl
from jax.experimental.pallas import tpu as pltpu, tpu_sc as plsc

N_NODES = 10000
DEG_PAD = 10240
N_EDGES = 320000
D = 128
NC, NS = 2, 16
NW = NC * NS
E_PER_W = N_EDGES // NW

_mesh = plsc.VectorSubcoreMesh(
    core_axis_name="c", subcore_axis_name="s", num_cores=NC, num_subcores=NS)


DBLK = 80
DNB = E_PER_W // DBLK
DEG_ROWS_PER_TILE = DEG_PAD // NS


@functools.partial(
    pl.kernel,
    out_type=jax.ShapeDtypeStruct((NC, DEG_PAD), jnp.float32),
    mesh=_mesh,
    scratch_types=[
        pltpu.VMEM((DNB, DBLK), jnp.int32),
        pltpu.VMEM((DBLK,), jnp.float32),
        pltpu.VMEM((DEG_ROWS_PER_TILE,), jnp.float32),
        pltpu.VMEM_SHARED((DEG_PAD,), jnp.float32),
    ],
)
def _sc_degree(ei_hbm, deg_hbm, idx_v, ones_v, zero_v, deg_sh):
    c = lax.axis_index("c")
    s = lax.axis_index("s")
    wid = s * NC + c

    for j in range(DBLK // 16):
        ones_v[pl.ds(j * 16, 16)] = jnp.full((16,), 1.0, jnp.float32)
    for j in range(DEG_ROWS_PER_TILE // 16):
        zero_v[pl.ds(j * 16, 16)] = jnp.zeros((16,), jnp.float32)
    pltpu.sync_copy(zero_v,
                    deg_sh.at[pl.ds(s * DEG_ROWS_PER_TILE, DEG_ROWS_PER_TILE)])
    plsc.subcore_barrier()

    pltpu.sync_copy(ei_hbm.at[1, wid], idx_v)

    def body(b, _):
        pltpu.sync_copy(ones_v, deg_sh.at[idx_v.at[b]], add=True)
        return ()

    lax.fori_loop(0, DNB, body, (), unroll=False)
    plsc.subcore_barrier()
    pltpu.sync_copy(deg_sh.at[pl.ds(s * DEG_ROWS_PER_TILE, DEG_ROWS_PER_TILE)],
                    deg_hbm.at[c, pl.ds(s * DEG_ROWS_PER_TILE,
                                        DEG_ROWS_PER_TILE)])


BLK = 40
NB = E_PER_W // BLK
NBUF = 5
WAVES = NB // NBUF
WPAIR = WAVES // 2
ROWS_PER_TILE = DEG_PAD // NS


@functools.partial(
    pl.kernel,
    out_type=jax.ShapeDtypeStruct((NC, DEG_PAD, D), jnp.float32),
    mesh=_mesh,
    scratch_types=[
        pltpu.VMEM((2, NBUF, BLK), jnp.int32),
        pltpu.VMEM((2, NBUF, BLK), jnp.int32),
        pltpu.VMEM((NBUF, BLK, D), jnp.float32),
        pltpu.VMEM_SHARED((DEG_PAD, D), jnp.float32),
        [pltpu.SemaphoreType.DMA] * NBUF,
        [pltpu.SemaphoreType.DMA] * NBUF,
        pltpu.SemaphoreType.DMA,
        pltpu.SemaphoreType.DMA,
    ],
)
def _sc_aggregate(idx_hbm, hs_hbm, out_hbm,
                  ixa_v, ixb_v, rows_v, agg_sh,
                  gsem, ssem, isema, isemb):
    c = lax.axis_index("c")
    s = lax.axis_index("s")
    wid = s * NC + c
    row0 = s * ROWS_PER_TILE

    for r in range(8):
        for l in range(D // 16):
            rows_v[0, r, pl.ds(l * 16, 16)] = jnp.zeros((16,), jnp.float32)

    def zero_body(t, _):
        pltpu.sync_copy(rows_v.at[0, pl.ds(0, 8)],
                        agg_sh.at[pl.ds(row0 + t * 8, 8)])
        return ()

    lax.fori_loop(0, ROWS_PER_TILE // 8, zero_body, (), unroll=False)
    plsc.subcore_barrier()

    def fetch_idx(v, ix_v, isem):
        pltpu.async_copy(idx_hbm.at[0, wid, v], ix_v.at[0], isem)
        pltpu.async_copy(idx_hbm.at[1, wid, v], ix_v.at[1], isem)

    def wait_idx(ix_v, isem):
        pltpu.make_async_copy(idx_hbm.at[0, wid, 0], ix_v.at[0], isem).wait()
        pltpu.make_async_copy(idx_hbm.at[1, wid, 0], ix_v.at[1], isem).wait()

    fetch_idx(0, ixa_v, isema)
    fetch_idx(1, ixb_v, isemb)
    wait_idx(ixa_v, isema)
    for j in range(NBUF):
        pltpu.async_copy(hs_hbm.at[ixa_v.at[0, j]], rows_v.at[j], gsem[j])

    def body(k, _):
        for j in range(NBUF):
            pltpu.make_async_copy(hs_hbm.at[ixa_v.at[0, j]], rows_v.at[j],
                                  gsem[j]).wait()
            pltpu.async_copy(rows_v.at[j], agg_sh.at[ixa_v.at[1, j]], ssem[j],
                             add=True)

        wait_idx(ixb_v, isemb)
        for j in range(NBUF):
            pltpu.make_async_copy(rows_v.at[j], agg_sh.at[ixa_v.at[1, j]],
                                  ssem[j]).wait()
            pltpu.async_copy(hs_hbm.at[ixb_v.at[0, j]], rows_v.at[j], gsem[j])

        @pl.when(k < WPAIR - 1)
        def _():
            fetch_idx(2 * k + 2, ixa_v, isema)

        for j in range(NBUF):
            pltpu.make_async_copy(hs_hbm.at[ixb_v.at[0, j]], rows_v.at[j],
                                  gsem[j]).wait()
            pltpu.async_copy(rows_v.at[j], agg_sh.at[ixb_v.at[1, j]], ssem[j],
                             add=True)

        @pl.when(k < WPAIR - 1)
        def _():
            wait_idx(ixa_v, isema)
        for j in range(NBUF):
            pltpu.make_async_copy(rows_v.at[j], agg_sh.at[ixb_v.at[1, j]],
                                  ssem[j]).wait()

            @pl.when(k < WPAIR - 1)
            def _():
                pltpu.async_copy(hs_hbm.at[ixa_v.at[0, j]], rows_v.at[j],
                                 gsem[j])

        @pl.when(k < WPAIR - 1)
        def _():
            fetch_idx(2 * k + 3, ixb_v, isemb)

        return ()

    lax.fori_loop(0, WPAIR, body, (), unroll=False)
    plsc.subcore_barrier()
    pltpu.sync_copy(agg_sh.at[pl.ds(row0, ROWS_PER_TILE)],
                    out_hbm.at[c, pl.ds(row0, ROWS_PER_TILE)])


_TC_ROWS = 1000
_TC_GRID = N_NODES // _TC_ROWS

_row_spec = pl.BlockSpec((_TC_ROWS, D), lambda i: (i, 0))
_w_spec = pl.BlockSpec((D, D), lambda i: (0, 0))
_b_spec = pl.BlockSpec((1, D), lambda i: (0, 0))
_p0_spec = pl.BlockSpec((1, _TC_ROWS, D), lambda i: (0, i, 0))
_p1_spec = pl.BlockSpec((1, _TC_ROWS, D), lambda i: (1, i, 0))


def _tc1a_body(x_ref, w_ref, h_ref):
    h_ref[...] = jnp.dot(x_ref[...], w_ref[...],
                         preferred_element_type=jnp.float32)


def _tc1a(x, w1):
    return pl.pallas_call(
        _tc1a_body,
        grid=(_TC_GRID,),
        in_specs=[_row_spec, _w_spec],
        out_specs=_row_spec,
        out_shape=jax.ShapeDtypeStruct((N_NODES, D), jnp.float32),
    )(x, w1)


def _tc1b_body(h_ref, d0_ref, d1_ref, hs_ref, dinv_ref):
    deg = d0_ref[0] + d1_ref[0] + 1.0
    dinv = lax.rsqrt(deg)
    hs_ref[...] = h_ref[...] * dinv
    dinv_ref[...] = dinv


def _tc1b(h1, db):
    return pl.pallas_call(
        _tc1b_body,
        grid=(_TC_GRID,),
        in_specs=[_row_spec, _p0_spec, _p1_spec],
        out_specs=[_row_spec, _row_spec],
        out_shape=[jax.ShapeDtypeStruct((N_NODES, D), jnp.float32),
                   jax.ShapeDtypeStruct((N_NODES, D), jnp.float32)],
    )(h1, db, db)


def _tc2_body(p0_ref, p1_ref, hs1_ref, dinv_ref, b1_ref, w_ref, hs2_ref):
    dinv = dinv_ref[...]
    z = ((p0_ref[0] + p1_ref[0] + hs1_ref[...]) * dinv + b1_ref[...])
    z = jnp.maximum(z, 0.0)
    h2 = jnp.dot(z, w_ref[...], preferred_element_type=jnp.float32)
    hs2_ref[...] = h2 * dinv


def _tc2(p, hs1, dinvb, b1, w2):
    return pl.pallas_call(
        _tc2_body,
        grid=(_TC_GRID,),
        in_specs=[_p0_spec, _p1_spec, _row_spec, _row_spec, _b_spec, _w_spec],
        out_specs=_row_spec,
        out_shape=jax.ShapeDtypeStruct((N_NODES, D), jnp.float32),
    )(p, p, hs1, dinvb, b1, w2)


def _tc3_body(q0_ref, q1_ref, hs2_ref, dinv_ref, b2_ref, out_ref):
    out_ref[...] = ((q0_ref[0] + q1_ref[0] + hs2_ref[...]) * dinv_ref[...]
                    + b2_ref[...])


def _tc3(q, hs2, dinvb, b2):
    return pl.pallas_call(
        _tc3_body,
        grid=(_TC_GRID,),
        in_specs=[_p0_spec, _p1_spec, _row_spec, _row_spec, _b_spec],
        out_specs=_row_spec,
        out_shape=jax.ShapeDtypeStruct((N_NODES, D), jnp.float32),
    )(q, q, hs2, dinvb, b2)


def kernel(x, edge_index, W1, b1, W2, b2):
    e = edge_index.astype(jnp.int32)
    ei = e.reshape(2, NW, WAVES, NBUF, BLK)
    eid = e.reshape(2, NW, DNB, DBLK)
    b1r = b1.reshape(1, D)
    b2r = b2.reshape(1, D)

    h1 = _tc1a(x, W1)
    degp = _sc_degree(eid)
    db = jnp.broadcast_to(degp[:, :N_NODES, None], (2, N_NODES, D))

    hs1, dinvb = _tc1b(h1, db)
    p = _sc_aggregate(ei, hs1)
    hs2 = _tc2(p, hs1, dinvb, b1r, W2)
    q = _sc_aggregate(ei, hs2)
    return _tc3(q, hs2, dinvb, b2r)


# speedup vs baseline: 31.3074x; 1.0126x over previous
import functools

import jax
import jax.numpy as jnp
from jax import lax
from jax.experimental import pallas as pl
from jax.experimental.pallas import tpu as pltpu, tpu_sc as plsc

N_NODES = 10000
DEG_PAD = 10240
N_EDGES = 320000
D = 128
NC, NS = 2, 16
NW = NC * NS
E_PER_W = N_EDGES // NW

_mesh = plsc.VectorSubcoreMesh(
    core_axis_name="c", subcore_axis_name="s", num_cores=NC, num_subcores=NS)


DEG_ROWS_PER_TILE = DEG_PAD // NS


BLK = 40
NB = E_PER_W // BLK
NBUF = 5
WAVES = NB // NBUF
WPAIR = WAVES // 2
ROWS_PER_TILE = DEG_PAD // NS


@functools.partial(
    pl.kernel,
    out_type=jax.ShapeDtypeStruct((NC, DEG_PAD, D), jnp.float32),
    mesh=_mesh,
    scratch_types=[
        pltpu.VMEM((2, NBUF, BLK), jnp.int32),
        pltpu.VMEM((2, NBUF, BLK), jnp.int32),
        pltpu.VMEM((NBUF, BLK, D), jnp.float32),
        pltpu.VMEM_SHARED((DEG_PAD, D), jnp.float32),
        [pltpu.SemaphoreType.DMA] * NBUF,
        [pltpu.SemaphoreType.DMA] * NBUF,
        pltpu.SemaphoreType.DMA,
        pltpu.SemaphoreType.DMA,
    ],
)
def _sc_aggregate(idx_hbm, hs_hbm, out_hbm,
                  ixa_v, ixb_v, rows_v, agg_sh,
                  gsem, ssem, isema, isemb):
    c = lax.axis_index("c")
    s = lax.axis_index("s")
    wid = s * NC + c
    row0 = s * ROWS_PER_TILE

    for r in range(8):
        for l in range(D // 16):
            rows_v[0, r, pl.ds(l * 16, 16)] = jnp.zeros((16,), jnp.float32)

    def zero_body(t, _):
        pltpu.sync_copy(rows_v.at[0, pl.ds(0, 8)],
                        agg_sh.at[pl.ds(row0 + t * 8, 8)])
        return ()

    lax.fori_loop(0, ROWS_PER_TILE // 8, zero_body, (), unroll=False)
    plsc.subcore_barrier()

    def fetch_idx(v, ix_v, isem):
        pltpu.async_copy(idx_hbm.at[0, wid, v], ix_v.at[0], isem)
        pltpu.async_copy(idx_hbm.at[1, wid, v], ix_v.at[1], isem)

    def wait_idx(ix_v, isem):
        pltpu.make_async_copy(idx_hbm.at[0, wid, 0], ix_v.at[0], isem).wait()
        pltpu.make_async_copy(idx_hbm.at[1, wid, 0], ix_v.at[1], isem).wait()

    fetch_idx(0, ixa_v, isema)
    fetch_idx(1, ixb_v, isemb)
    wait_idx(ixa_v, isema)
    for j in range(NBUF):
        pltpu.async_copy(hs_hbm.at[ixa_v.at[0, j]], rows_v.at[j], gsem[j])

    def body(k, _):
        for j in range(NBUF):
            pltpu.make_async_copy(hs_hbm.at[ixa_v.at[0, j]], rows_v.at[j],
                                  gsem[j]).wait()
            pltpu.async_copy(rows_v.at[j], agg_sh.at[ixa_v.at[1, j]], ssem[j],
                             add=True)

        wait_idx(ixb_v, isemb)
        for j in range(NBUF):
            pltpu.make_async_copy(rows_v.at[j], agg_sh.at[ixa_v.at[1, j]],
                                  ssem[j]).wait()
            pltpu.async_copy(hs_hbm.at[ixb_v.at[0, j]], rows_v.at[j], gsem[j])

        @pl.when(k < WPAIR - 1)
        def _():
            fetch_idx(2 * k + 2, ixa_v, isema)

        for j in range(NBUF):
            pltpu.make_async_copy(hs_hbm.at[ixb_v.at[0, j]], rows_v.at[j],
                                  gsem[j]).wait()
            pltpu.async_copy(rows_v.at[j], agg_sh.at[ixb_v.at[1, j]], ssem[j],
                             add=True)

        @pl.when(k < WPAIR - 1)
        def _():
            wait_idx(ixa_v, isema)
        for j in range(NBUF):
            pltpu.make_async_copy(rows_v.at[j], agg_sh.at[ixb_v.at[1, j]],
                                  ssem[j]).wait()

            @pl.when(k < WPAIR - 1)
            def _():
                pltpu.async_copy(hs_hbm.at[ixa_v.at[0, j]], rows_v.at[j],
                                 gsem[j])

        @pl.when(k < WPAIR - 1)
        def _():
            fetch_idx(2 * k + 3, ixb_v, isemb)

        return ()

    lax.fori_loop(0, WPAIR, body, (), unroll=False)
    plsc.subcore_barrier()
    pltpu.sync_copy(agg_sh.at[pl.ds(row0, ROWS_PER_TILE)],
                    out_hbm.at[c, pl.ds(row0, ROWS_PER_TILE)])


@functools.partial(
    pl.kernel,
    out_type=jax.ShapeDtypeStruct((NC, DEG_PAD), jnp.float32),
    mesh=_mesh,
    scratch_types=[
        pltpu.VMEM((WAVES, NBUF, BLK), jnp.int32),
        pltpu.VMEM((48,), jnp.float32),
        pltpu.VMEM((DEG_ROWS_PER_TILE,), jnp.float32),
        pltpu.VMEM_SHARED((DEG_PAD,), jnp.float32),
        pltpu.SemaphoreType.DMA,
    ],
)
def _sc_degree(ei_hbm, deg_hbm, idx_v, ones_v, zero_v, deg_sh, dsem):
    c = lax.axis_index("c")
    s = lax.axis_index("s")
    wid = s * NC + c

    for j in range(48 // 16):
        ones_v[pl.ds(j * 16, 16)] = jnp.full((16,), 1.0, jnp.float32)
    for j in range(DEG_ROWS_PER_TILE // 16):
        zero_v[pl.ds(j * 16, 16)] = jnp.zeros((16,), jnp.float32)
    pltpu.sync_copy(zero_v,
                    deg_sh.at[pl.ds(s * DEG_ROWS_PER_TILE, DEG_ROWS_PER_TILE)])
    plsc.subcore_barrier()

    pltpu.sync_copy(ei_hbm.at[1, wid], idx_v)
    ones40 = ones_v.at[pl.ds(0, BLK)]

    def body(v, _):
        for j in range(NBUF):
            pltpu.async_copy(ones40, deg_sh.at[idx_v.at[v, j]], dsem,
                             add=True)
        for j in range(NBUF):
            pltpu.make_async_copy(ones40, deg_sh.at[idx_v.at[v, 0]],
                                  dsem).wait()
        return ()

    lax.fori_loop(0, WAVES, body, (), unroll=False)
    plsc.subcore_barrier()
    pltpu.sync_copy(deg_sh.at[pl.ds(s * DEG_ROWS_PER_TILE, DEG_ROWS_PER_TILE)],
                    deg_hbm.at[c, pl.ds(s * DEG_ROWS_PER_TILE,
                                        DEG_ROWS_PER_TILE)])


_TC_ROWS = 1000
_TC_GRID = N_NODES // _TC_ROWS

_row_spec = pl.BlockSpec((_TC_ROWS, D), lambda i: (i, 0))
_w_spec = pl.BlockSpec((D, D), lambda i: (0, 0))
_b_spec = pl.BlockSpec((1, D), lambda i: (0, 0))
_p0_spec = pl.BlockSpec((1, _TC_ROWS, D), lambda i: (0, i, 0))
_p1_spec = pl.BlockSpec((1, _TC_ROWS, D), lambda i: (1, i, 0))


def _tc1a_body(x_ref, w_ref, h_ref):
    h_ref[...] = jnp.dot(x_ref[...], w_ref[...],
                         preferred_element_type=jnp.float32)


def _tc1a(x, w1):
    return pl.pallas_call(
        _tc1a_body,
        grid=(_TC_GRID,),
        in_specs=[_row_spec, _w_spec],
        out_specs=_row_spec,
        out_shape=jax.ShapeDtypeStruct((N_NODES, D), jnp.float32),
    )(x, w1)


def _tc1b_body(h_ref, d0_ref, d1_ref, hs_ref, dinv_ref):
    deg = d0_ref[0] + d1_ref[0] + 1.0
    dinv = lax.rsqrt(deg)
    hs_ref[...] = h_ref[...] * dinv
    dinv_ref[...] = dinv


def _tc1b(h1, db):
    return pl.pallas_call(
        _tc1b_body,
        grid=(_TC_GRID,),
        in_specs=[_row_spec, _p0_spec, _p1_spec],
        out_specs=[_row_spec, _row_spec],
        out_shape=[jax.ShapeDtypeStruct((N_NODES, D), jnp.float32),
                   jax.ShapeDtypeStruct((N_NODES, D), jnp.float32)],
    )(h1, db, db)


def _tc2_body(p0_ref, p1_ref, hs1_ref, dinv_ref, b1_ref, w_ref, hs2_ref):
    dinv = dinv_ref[...]
    z = ((p0_ref[0] + p1_ref[0] + hs1_ref[...]) * dinv + b1_ref[...])
    z = jnp.maximum(z, 0.0)
    h2 = jnp.dot(z, w_ref[...], preferred_element_type=jnp.float32)
    hs2_ref[...] = h2 * dinv


def _tc2(p, hs1, dinvb, b1, w2):
    return pl.pallas_call(
        _tc2_body,
        grid=(_TC_GRID,),
        in_specs=[_p0_spec, _p1_spec, _row_spec, _row_spec, _b_spec, _w_spec],
        out_specs=_row_spec,
        out_shape=jax.ShapeDtypeStruct((N_NODES, D), jnp.float32),
    )(p, p, hs1, dinvb, b1, w2)


def _tc3_body(q0_ref, q1_ref, hs2_ref, dinv_ref, b2_ref, out_ref):
    out_ref[...] = ((q0_ref[0] + q1_ref[0] + hs2_ref[...]) * dinv_ref[...]
                    + b2_ref[...])


def _tc3(q, hs2, dinvb, b2):
    return pl.pallas_call(
        _tc3_body,
        grid=(_TC_GRID,),
        in_specs=[_p0_spec, _p1_spec, _row_spec, _row_spec, _b_spec],
        out_specs=_row_spec,
        out_shape=jax.ShapeDtypeStruct((N_NODES, D), jnp.float32),
    )(q, q, hs2, dinvb, b2)


def kernel(x, edge_index, W1, b1, W2, b2):
    e = edge_index.astype(jnp.int32)
    ei = e.reshape(2, NW, WAVES, NBUF, BLK)
    b1r = b1.reshape(1, D)
    b2r = b2.reshape(1, D)

    h1 = _tc1a(x, W1)
    degp = _sc_degree(ei)
    db = jnp.broadcast_to(degp[:, :N_NODES, None], (2, N_NODES, D))

    hs1, dinvb = _tc1b(h1, db)
    p = _sc_aggregate(ei, hs1)
    hs2 = _tc2(p, hs1, dinvb, b1r, W2)
    q = _sc_aggregate(ei, hs2)
    return _tc3(q, hs2, dinvb, b2r)
